# baseline (speedup 1.0000x reference)
import jax
import jax.numpy as jnp
from jax import lax
from jax.experimental import pallas as pl
from jax.experimental.pallas import tpu as pltpu
from jax.experimental.pallas import tpu_sc as plsc

VOCAB_SIZE = 100000
MODEL_DIM = 1024
MAXLEN = 2048
BATCH = 4
SEQ = 2048

NUM_CORES = 2
NUM_SUBCORES = 16
NUM_WORKERS = NUM_CORES * NUM_SUBCORES
LANES = 16

SPW = SEQ // NUM_WORKERS
CHUNK = 32
NCHUNK = SPW // CHUNK


def _body(x_hbm, tok_hbm, pos_hbm, out_hbm, pos_v, idx_v, tok_v, gsem):
    wid = lax.axis_index("s") * NUM_CORES + lax.axis_index("c")
    s0 = wid * SPW

    pltpu.sync_copy(pos_hbm.at[pl.ds(s0, SPW)], pos_v)

    for b in range(BATCH):
        pltpu.sync_copy(x_hbm.at[pl.ds(b * SEQ + s0, SPW)], idx_v)
        for c in range(NCHUNK):
            pltpu.async_copy(
                tok_hbm.at[idx_v.at[pl.ds(c * CHUNK, CHUNK)]], tok_v, gsem
            ).wait()

            def add_row(r, _, c=c):
                def add_col(j, _, r=r):
                    sl = pl.ds(j * LANES, LANES)
                    tok_v[r, sl] = tok_v[r, sl] + pos_v[c * CHUNK + r, sl]
                    return 0

                return lax.fori_loop(0, MODEL_DIM // LANES, add_col, 0)

            lax.fori_loop(0, CHUNK, add_row, 0)
            pltpu.sync_copy(
                tok_v, out_hbm.at[pl.ds(b * SEQ + s0 + c * CHUNK, CHUNK)]
            )


@jax.jit
def _embed(xf, token_table, pos_table):
    mesh = plsc.VectorSubcoreMesh(core_axis_name="c", subcore_axis_name="s")
    return pl.kernel(
        _body,
        out_type=jax.ShapeDtypeStruct((BATCH * SEQ, MODEL_DIM), jnp.float32),
        mesh=mesh,
        scratch_types=[
            pltpu.VMEM((SPW, MODEL_DIM), jnp.float32),
            pltpu.VMEM((SPW,), jnp.int32),
            pltpu.VMEM((CHUNK, MODEL_DIM), jnp.float32),
            pltpu.SemaphoreType.DMA,
        ],
    )(xf, token_table, pos_table)


def kernel(x, token_table, pos_table):
    xf = x.reshape(-1).astype(jnp.int32)
    out = _embed(xf, token_table, pos_table)
    return out.reshape(x.shape[0], x.shape[1], MODEL_DIM)


# speedup vs baseline: 2.0294x; 2.0294x over previous
import jax
import jax.numpy as jnp
from jax import lax
from jax.experimental import pallas as pl
from jax.experimental.pallas import tpu as pltpu
from jax.experimental.pallas import tpu_sc as plsc

VOCAB_SIZE = 100000
MODEL_DIM = 1024
MAXLEN = 2048
BATCH = 4
SEQ = 2048

NUM_CORES = 2
NUM_SUBCORES = 16
NUM_WORKERS = NUM_CORES * NUM_SUBCORES
LANES = 16

SPW = SEQ // NUM_WORKERS
CHUNK = 32
NHALF = SPW // CHUNK
NITER = NHALF * BATCH


def _body(x_hbm, tok_hbm, pos_hbm, out_hbm,
          pos_v, idx_v, tok0, tok1, gsem0, gsem1, osem0, osem1):
    wid = lax.axis_index("s") * NUM_CORES + lax.axis_index("c")
    s0 = wid * SPW

    iw = [
        pltpu.async_copy(x_hbm.at[b, pl.ds(s0, SPW)], idx_v.at[b], gsem0)
        for b in range(BATCH)
    ]
    for w in iw:
        w.wait()

    toks = (tok0, tok1)
    gsems = (gsem0, gsem1)
    osems = (osem0, osem1)

    def start_gather(it):
        h, b = divmod(it, BATCH)
        p = it % 2
        return pltpu.async_copy(
            tok_hbm.at[idx_v.at[b, pl.ds(h * CHUNK, CHUNK)]], toks[p], gsems[p])

    def start_out(it):
        h, b = divmod(it, BATCH)
        p = it % 2
        return pltpu.async_copy(
            toks[p], out_hbm.at[b, pl.ds(s0 + h * CHUNK, CHUNK)], osems[p])

    g = [None] * NITER
    o = [None] * NITER
    g[0] = start_gather(0)
    for h in range(NHALF):
        pltpu.sync_copy(pos_hbm.at[pl.ds(s0 + h * CHUNK, CHUNK)], pos_v)
        for b in range(BATCH):
            it = h * BATCH + b
            p = it % 2
            g[it].wait()
            if it + 1 < NITER:
                if it >= 1:
                    o[it - 1].wait()
                g[it + 1] = start_gather(it + 1)

            tok = toks[p]

            def add_row(r, _, tok=tok):
                for j in range(MODEL_DIM // LANES):
                    sl = pl.ds(j * LANES, LANES)
                    tok[r, sl] = tok[r, sl] + pos_v[r, sl]
                return 0

            lax.fori_loop(0, CHUNK, add_row, 0)
            o[it] = start_out(it)
    o[NITER - 2].wait()
    o[NITER - 1].wait()


@jax.jit
def _embed(x, token_table, pos_table):
    mesh = plsc.VectorSubcoreMesh(core_axis_name="c", subcore_axis_name="s")
    return pl.kernel(
        _body,
        out_type=jax.ShapeDtypeStruct((BATCH, SEQ, MODEL_DIM), jnp.float32),
        mesh=mesh,
        scratch_types=[
            pltpu.VMEM((CHUNK, MODEL_DIM), jnp.float32),
            pltpu.VMEM((BATCH, SPW), jnp.int32),
            pltpu.VMEM((CHUNK, MODEL_DIM), jnp.float32),
            pltpu.VMEM((CHUNK, MODEL_DIM), jnp.float32),
            pltpu.SemaphoreType.DMA,
            pltpu.SemaphoreType.DMA,
            pltpu.SemaphoreType.DMA,
            pltpu.SemaphoreType.DMA,
        ],
    )(x, token_table, pos_table)


def kernel(x, token_table, pos_table):
    return _embed(x.astype(jnp.int32), token_table, pos_table)


# speedup vs baseline: 2.2264x; 1.0971x over previous
import jax
import jax.numpy as jnp
from jax import lax
from jax.experimental import pallas as pl
from jax.experimental.pallas import tpu as pltpu
from jax.experimental.pallas import tpu_sc as plsc

VOCAB_SIZE = 100000
MODEL_DIM = 1024
MAXLEN = 2048
BATCH = 4
SEQ = 2048

NUM_CORES = 2
NUM_SUBCORES = 16
NUM_WORKERS = NUM_CORES * NUM_SUBCORES
LANES = 16

SPW = SEQ // NUM_WORKERS
CHUNK = 16
NQ = SPW // CHUNK
NITER = NQ * BATCH
NBUF = 5
LOOKAHEAD = NBUF - 1


def _body(x_hbm, tok_hbm, pos_hbm, out_hbm,
          pos0, pos1, idx_v, t0, t1, t2, t3, t4,
          g0, g1, g2, g3, g4, os0, os1, os2, os3, os4, ps0, ps1, isem):
    wid = lax.axis_index("s") * NUM_CORES + lax.axis_index("c")
    s0 = wid * SPW

    toks = (t0, t1, t2, t3, t4)
    gsems = (g0, g1, g2, g3, g4)
    osems = (os0, os1, os2, os3, os4)
    pbufs = (pos0, pos1)
    psems = (ps0, ps1)

    iw = [
        pltpu.async_copy(x_hbm.at[b, pl.ds(s0, SPW)], idx_v.at[b], isem)
        for b in range(BATCH)
    ]

    def start_pos(q):
        return pltpu.async_copy(
            pos_hbm.at[pl.ds(s0 + q * CHUNK, CHUNK)], pbufs[q % 2], psems[q % 2])

    def start_gather(it):
        q, b = divmod(it, BATCH)
        p = it % NBUF
        return pltpu.async_copy(
            tok_hbm.at[idx_v.at[b, pl.ds(q * CHUNK, CHUNK)]], toks[p], gsems[p])

    def start_out(it):
        q, b = divmod(it, BATCH)
        p = it % NBUF
        return pltpu.async_copy(
            toks[p], out_hbm.at[b, pl.ds(s0 + q * CHUNK, CHUNK)], osems[p])

    pw = [None] * NQ
    pw[0] = start_pos(0)
    for w in iw:
        w.wait()

    g = [None] * NITER
    o = [None] * NITER
    for it in range(LOOKAHEAD):
        g[it] = start_gather(it)

    for it in range(NITER):
        q, b = divmod(it, BATCH)
        p = it % NBUF
        if b == 0:
            pw[q].wait()
            if q + 1 < NQ:
                pw[q + 1] = start_pos(q + 1)
        g[it].wait()

        tok = toks[p]
        pos = pbufs[q % 2]

        def add_row(r, _, tok=tok, pos=pos):
            for j in range(MODEL_DIM // LANES):
                sl = pl.ds(j * LANES, LANES)
                tok[r, sl] = tok[r, sl] + pos[r, sl]
            return 0

        lax.fori_loop(0, CHUNK, add_row, 0)
        o[it] = start_out(it)

        nxt = it + LOOKAHEAD
        if nxt < NITER:
            prev = nxt - NBUF
            if prev >= 0:
                o[prev].wait()
            g[nxt] = start_gather(nxt)

    for it in range(NITER - NBUF, NITER):
        if o[it] is not None:
            o[it].wait()


@jax.jit
def _embed(x, token_table, pos_table):
    mesh = plsc.VectorSubcoreMesh(core_axis_name="c", subcore_axis_name="s")
    return pl.kernel(
        _body,
        out_type=jax.ShapeDtypeStruct((BATCH, SEQ, MODEL_DIM), jnp.float32),
        mesh=mesh,
        scratch_types=(
            [pltpu.VMEM((CHUNK, MODEL_DIM), jnp.float32)] * 2
            + [pltpu.VMEM((BATCH, SPW), jnp.int32)]
            + [pltpu.VMEM((CHUNK, MODEL_DIM), jnp.float32)] * NBUF
            + [pltpu.SemaphoreType.DMA] * (2 * NBUF + 3)
        ),
    )(x, token_table, pos_table)


def kernel(x, token_table, pos_table):
    return _embed(x.astype(jnp.int32), token_table, pos_table)
